# Initial kernel scaffold; baseline (speedup 1.0000x reference)
import jax
import jax.numpy as jnp
from jax.experimental import pallas as pl


def kernel(img, b0_w_t, b0_b, b1_w_t, b1_gamma, b1_beta, b2_w_t, b2_gamma, b2_beta, b3_w_t, b3_gamma, b3_beta, b4_w_t, b4_gamma, b4_beta, final_wf2, final_b):
    raise NotImplementedError("write your pallas kernel here")


# speedup vs baseline: 1.6336x; 1.6336x over previous
import functools

import jax
import jax.numpy as jnp
from jax.experimental import pallas as pl
from jax.experimental.pallas import tpu as pltpu

_SLOPE = 0.2
_EPS = 1e-5


def _im2col(x):
    C, N, H, W = x.shape
    xp = jnp.pad(x, ((0, 0), (0, 0), (1, 1), (1, 1)))
    Ho, Wo = H // 2, W // 2
    taps = [xp[:, :, i:i + 2 * Ho:2, j:j + 2 * Wo:2]
            for i in range(4) for j in range(4)]
    p = jnp.stack(taps, axis=1)
    return p.reshape(C * 16, N * Ho * Wo).astype(jnp.bfloat16), Ho, Wo


def _repack_w(w_t):
    O, K = w_t.shape
    I = K // 16
    return w_t.reshape(O, 4, 4, I).transpose(0, 3, 1, 2).reshape(O, K)


def _pick_tile(n, hw):
    best = n
    for g in range(1, n + 1):
        if n % g == 0 and (g * hw) % 128 == 0:
            if g * hw <= 4096:
                best = g
            else:
                break
    if (best * hw) % 128 != 0:
        best = n
    return best


def _l0_body(w_ref, b_ref, p_ref, o_ref):
    w = w_ref[...].astype(jnp.bfloat16)
    y = jnp.dot(w, p_ref[...], preferred_element_type=jnp.float32)
    y = y + b_ref[...]
    y = jnp.where(y > 0, y, _SLOPE * y)
    o_ref[...] = y.astype(jnp.bfloat16)


def _layer0(w, b, patches, mt):
    C, K = w.shape
    M = patches.shape[1]
    grid = (M // mt,)
    return pl.pallas_call(
        _l0_body,
        out_shape=jax.ShapeDtypeStruct((C, M), jnp.bfloat16),
        grid=grid,
        in_specs=[
            pl.BlockSpec((C, K), lambda m: (0, 0)),
            pl.BlockSpec((C, 1), lambda m: (0, 0)),
            pl.BlockSpec((K, mt), lambda m: (0, m)),
        ],
        out_specs=pl.BlockSpec((C, mt), lambda m: (0, m)),
        compiler_params=pltpu.CompilerParams(
            dimension_semantics=("parallel",)),
    )(w, b, patches)


def _bn_body(w_ref, g_ref, b_ref, p_ref, o_ref, y_scr, sc_ref, sh_ref,
             *, mt, inv_m):
    p_id = pl.program_id(1)
    m = pl.program_id(2)

    @pl.when(p_id == 0)
    def _():
        w = w_ref[0].astype(jnp.bfloat16)
        y_scr[:, pl.ds(m * mt, mt)] = jnp.dot(
            w, p_ref[...], preferred_element_type=jnp.float32)
        o_ref[...] = jnp.zeros_like(o_ref)

    @pl.when((p_id == 1) & (m == 0))
    def _():
        y = y_scr[...]
        mean = jnp.sum(y, axis=1, keepdims=True) * inv_m
        d = y - mean
        var = jnp.sum(d * d, axis=1, keepdims=True) * inv_m
        scale = g_ref[0] * jax.lax.rsqrt(var + _EPS)
        sc_ref[...] = scale
        sh_ref[...] = b_ref[0] - mean * scale

    @pl.when(p_id == 1)
    def _():
        z = y_scr[:, pl.ds(m * mt, mt)] * sc_ref[...] + sh_ref[...]
        z = jnp.where(z > 0, z, _SLOPE * z)
        o_ref[...] = z[None].astype(jnp.bfloat16)


def _bn_layer(w, gamma, beta, patches, mt):
    C, K = w.shape
    M = patches.shape[1]
    Ch = C // 2
    grid = (2, 2, M // mt)
    w3 = w.reshape(2, Ch, K)
    g3 = gamma.reshape(2, Ch, 1)
    b3 = beta.reshape(2, Ch, 1)
    body = functools.partial(_bn_body, mt=mt, inv_m=1.0 / M)
    y = pl.pallas_call(
        body,
        out_shape=jax.ShapeDtypeStruct((2, Ch, M), jnp.bfloat16),
        grid=grid,
        in_specs=[
            pl.BlockSpec((1, Ch, K), lambda c, p, m: (c, 0, 0)),
            pl.BlockSpec((1, Ch, 1), lambda c, p, m: (c, 0, 0)),
            pl.BlockSpec((1, Ch, 1), lambda c, p, m: (c, 0, 0)),
            pl.BlockSpec((K, mt),
                         lambda c, p, m: (0, jnp.where(p == 0, m, 0))),
        ],
        out_specs=pl.BlockSpec((1, Ch, mt), lambda c, p, m: (c, 0, m)),
        scratch_shapes=[
            pltpu.VMEM((Ch, M), jnp.float32),
            pltpu.VMEM((Ch, 1), jnp.float32),
            pltpu.VMEM((Ch, 1), jnp.float32),
        ],
        compiler_params=pltpu.CompilerParams(
            dimension_semantics=("parallel", "arbitrary", "arbitrary")),
    )(w3, g3, b3, patches)
    return y.reshape(C, M)


def _tail_body(w_ref, g_ref, b_ref, p_ref, wf_ref, bf_ref, o_ref, *, n, inv_m):
    w = w_ref[...].astype(jnp.bfloat16)
    y = jnp.dot(w, p_ref[...], preferred_element_type=jnp.float32)
    mean = jnp.sum(y, axis=1, keepdims=True) * inv_m
    d = y - mean
    var = jnp.sum(d * d, axis=1, keepdims=True) * inv_m
    scale = g_ref[...] * jax.lax.rsqrt(var + _EPS)
    z = d * scale + b_ref[...]
    z = jnp.where(z > 0, z, _SLOPE * z)
    acc = bf_ref[...]
    for j in range(16):
        acc = acc + jnp.dot(wf_ref[j:j + 1, :], z[:, j * n:(j + 1) * n],
                            preferred_element_type=jnp.float32)
    o_ref[...] = acc


def _tail(w, gamma, beta, patches, wf_t, bfin, n):
    C, K = w.shape
    M = patches.shape[1]
    body = functools.partial(_tail_body, n=n, inv_m=1.0 / M)
    return pl.pallas_call(
        body,
        out_shape=jax.ShapeDtypeStruct((1, n), jnp.float32),
        grid=(1,),
        in_specs=[
            pl.BlockSpec((C, K), lambda i: (0, 0)),
            pl.BlockSpec((C, 1), lambda i: (0, 0)),
            pl.BlockSpec((C, 1), lambda i: (0, 0)),
            pl.BlockSpec((K, M), lambda i: (0, 0)),
            pl.BlockSpec((16, C), lambda i: (0, 0)),
            pl.BlockSpec((1, 1), lambda i: (0, 0)),
        ],
        out_specs=pl.BlockSpec((1, n), lambda i: (0, 0)),
        compiler_params=pltpu.CompilerParams(
            dimension_semantics=("arbitrary",)),
    )(w, gamma, beta, patches, wf_t, bfin)


def kernel(img, b0_w_t, b0_b, b1_w_t, b1_gamma, b1_beta,
           b2_w_t, b2_gamma, b2_beta, b3_w_t, b3_gamma, b3_beta,
           b4_w_t, b4_gamma, b4_beta, final_wf2, final_b):
    N = img.shape[0]
    x = jnp.transpose(img, (1, 0, 2, 3)).astype(jnp.bfloat16)

    p, Ho, Wo = _im2col(x)
    mt = _pick_tile(N, Ho * Wo) * Ho * Wo
    y = _layer0(_repack_w(b0_w_t), b0_b, p, mt)
    x = y.reshape(8, N, Ho, Wo)

    for w_t, ga, be in ((b1_w_t, b1_gamma, b1_beta),
                        (b2_w_t, b2_gamma, b2_beta),
                        (b3_w_t, b3_gamma, b3_beta)):
        p, Ho, Wo = _im2col(x)
        mt = _pick_tile(N, Ho * Wo) * Ho * Wo
        y = _bn_layer(_repack_w(w_t), ga, be, p, mt)
        x = y.reshape(w_t.shape[0], N, Ho, Wo)

    C, _, H, W = x.shape
    xp = jnp.pad(x, ((0, 0), (0, 0), (1, 1), (1, 1)))
    Ho, Wo = H // 2, W // 2
    taps = [xp[:, :, i:i + 2 * Ho:2, j:j + 2 * Wo:2]
            for i in range(4) for j in range(4)]
    p = jnp.stack(taps, axis=1)
    p = p.transpose(0, 1, 3, 4, 2)
    p = p.reshape(C * 16, Ho * Wo * N).astype(jnp.bfloat16)
    out = _tail(_repack_w(b4_w_t), b4_gamma, b4_beta, p,
                final_wf2.T, final_b, N)
    return out.reshape(N, 1, 1, 1).astype(jnp.float32)


# speedup vs baseline: 13.9051x; 8.5117x over previous
import functools

import jax
import jax.numpy as jnp
from jax.experimental import pallas as pl
from jax.experimental.pallas import tpu as pltpu

_SLOPE = 0.2
_EPS = 1e-5


def _im2col(x):
    C, H, W, N = x.shape
    xp = jnp.pad(x, ((0, 0), (1, 1), (1, 1), (0, 0)))
    Hh, Wh = (H + 2) // 2, (W + 2) // 2
    ph = xp.reshape(C, Hh, 2, Wh, 2, N).transpose(2, 4, 0, 1, 3, 5)
    Ho, Wo = H // 2, W // 2
    taps = [ph[i & 1, j & 1, :, (i >> 1):(i >> 1) + Ho,
               (j >> 1):(j >> 1) + Wo, :]
            for i in range(4) for j in range(4)]
    p = jnp.stack(taps, axis=1)
    return p.reshape(C * 16, Ho * Wo * N), Ho, Wo


def _repack_w(w_t):
    O, K = w_t.shape
    I = K // 16
    return w_t.reshape(O, 4, 4, I).transpose(0, 3, 1, 2).reshape(O, K)


def _pick_tile(ho, won):
    g = 1
    while g < ho and ((g * won) % 128 != 0 or g * won < 2048):
        g *= 2
    return g * won


def _l0_body(w_ref, b_ref, p_ref, o_ref):
    w = w_ref[...].astype(jnp.bfloat16)
    y = jnp.dot(w, p_ref[...], preferred_element_type=jnp.float32)
    y = y + b_ref[...]
    y = jnp.where(y > 0, y, _SLOPE * y)
    o_ref[...] = y.astype(jnp.bfloat16)


def _layer0(w, b, patches, mt):
    C, K = w.shape
    M = patches.shape[1]
    grid = (M // mt,)
    return pl.pallas_call(
        _l0_body,
        out_shape=jax.ShapeDtypeStruct((C, M), jnp.bfloat16),
        grid=grid,
        in_specs=[
            pl.BlockSpec((C, K), lambda m: (0, 0)),
            pl.BlockSpec((C, 1), lambda m: (0, 0)),
            pl.BlockSpec((K, mt), lambda m: (0, m)),
        ],
        out_specs=pl.BlockSpec((C, mt), lambda m: (0, m)),
        compiler_params=pltpu.CompilerParams(
            dimension_semantics=("parallel",)),
    )(w, b, patches)


def _bn_body(w_ref, g_ref, b_ref, p_ref, o_ref, y_scr, sc_ref, sh_ref,
             *, mt, inv_m):
    p_id = pl.program_id(1)
    m = pl.program_id(2)

    @pl.when(p_id == 0)
    def _():
        w = w_ref[0].astype(jnp.bfloat16)
        y_scr[:, pl.ds(m * mt, mt)] = jnp.dot(
            w, p_ref[...], preferred_element_type=jnp.float32)
        o_ref[...] = jnp.zeros_like(o_ref)

    @pl.when((p_id == 1) & (m == 0))
    def _():
        y = y_scr[...]
        mean = jnp.sum(y, axis=1, keepdims=True) * inv_m
        d = y - mean
        var = jnp.sum(d * d, axis=1, keepdims=True) * inv_m
        scale = g_ref[0] * jax.lax.rsqrt(var + _EPS)
        sc_ref[...] = scale
        sh_ref[...] = b_ref[0] - mean * scale

    @pl.when(p_id == 1)
    def _():
        z = y_scr[:, pl.ds(m * mt, mt)] * sc_ref[...] + sh_ref[...]
        z = jnp.where(z > 0, z, _SLOPE * z)
        o_ref[...] = z[None].astype(jnp.bfloat16)


def _bn_layer(w, gamma, beta, patches, mt):
    C, K = w.shape
    M = patches.shape[1]
    Ch = C // 2
    grid = (2, 2, M // mt)
    w3 = w.reshape(2, Ch, K)
    g3 = gamma.reshape(2, Ch, 1)
    b3 = beta.reshape(2, Ch, 1)
    body = functools.partial(_bn_body, mt=mt, inv_m=1.0 / M)
    y = pl.pallas_call(
        body,
        out_shape=jax.ShapeDtypeStruct((2, Ch, M), jnp.bfloat16),
        grid=grid,
        in_specs=[
            pl.BlockSpec((1, Ch, K), lambda c, p, m: (c, 0, 0)),
            pl.BlockSpec((1, Ch, 1), lambda c, p, m: (c, 0, 0)),
            pl.BlockSpec((1, Ch, 1), lambda c, p, m: (c, 0, 0)),
            pl.BlockSpec((K, mt),
                         lambda c, p, m: (0, jnp.where(p == 0, m, 0))),
        ],
        out_specs=pl.BlockSpec((1, Ch, mt), lambda c, p, m: (c, 0, m)),
        scratch_shapes=[
            pltpu.VMEM((Ch, M), jnp.float32),
            pltpu.VMEM((Ch, 1), jnp.float32),
            pltpu.VMEM((Ch, 1), jnp.float32),
        ],
        compiler_params=pltpu.CompilerParams(
            dimension_semantics=("parallel", "arbitrary", "arbitrary")),
    )(w3, g3, b3, patches)
    return y.reshape(C, M)


def _tail_body(w_ref, g_ref, b_ref, p_ref, wf_ref, bf_ref, o_ref, *, n, inv_m):
    w = w_ref[...].astype(jnp.bfloat16)
    y = jnp.dot(w, p_ref[...], preferred_element_type=jnp.float32)
    mean = jnp.sum(y, axis=1, keepdims=True) * inv_m
    d = y - mean
    var = jnp.sum(d * d, axis=1, keepdims=True) * inv_m
    scale = g_ref[...] * jax.lax.rsqrt(var + _EPS)
    z = d * scale + b_ref[...]
    z = jnp.where(z > 0, z, _SLOPE * z)
    acc = bf_ref[...]
    for j in range(16):
        acc = acc + jnp.dot(wf_ref[j:j + 1, :], z[:, j * n:(j + 1) * n],
                            preferred_element_type=jnp.float32)
    o_ref[...] = acc


def _tail(w, gamma, beta, patches, wf_t, bfin, n):
    C, K = w.shape
    M = patches.shape[1]
    body = functools.partial(_tail_body, n=n, inv_m=1.0 / M)
    return pl.pallas_call(
        body,
        out_shape=jax.ShapeDtypeStruct((1, n), jnp.float32),
        grid=(1,),
        in_specs=[
            pl.BlockSpec((C, K), lambda i: (0, 0)),
            pl.BlockSpec((C, 1), lambda i: (0, 0)),
            pl.BlockSpec((C, 1), lambda i: (0, 0)),
            pl.BlockSpec((K, M), lambda i: (0, 0)),
            pl.BlockSpec((16, C), lambda i: (0, 0)),
            pl.BlockSpec((1, 1), lambda i: (0, 0)),
        ],
        out_specs=pl.BlockSpec((1, n), lambda i: (0, 0)),
        compiler_params=pltpu.CompilerParams(
            dimension_semantics=("arbitrary",)),
    )(w, gamma, beta, patches, wf_t, bfin)


def kernel(img, b0_w_t, b0_b, b1_w_t, b1_gamma, b1_beta,
           b2_w_t, b2_gamma, b2_beta, b3_w_t, b3_gamma, b3_beta,
           b4_w_t, b4_gamma, b4_beta, final_wf2, final_b):
    N = img.shape[0]
    x = jnp.transpose(img, (1, 2, 3, 0)).astype(jnp.bfloat16)

    p, Ho, Wo = _im2col(x)
    mt = _pick_tile(Ho, Wo * N)
    y = _layer0(_repack_w(b0_w_t), b0_b, p, mt)
    x = y.reshape(8, Ho, Wo, N)

    for w_t, ga, be in ((b1_w_t, b1_gamma, b1_beta),
                        (b2_w_t, b2_gamma, b2_beta),
                        (b3_w_t, b3_gamma, b3_beta)):
        p, Ho, Wo = _im2col(x)
        mt = _pick_tile(Ho, Wo * N)
        y = _bn_layer(_repack_w(w_t), ga, be, p, mt)
        x = y.reshape(w_t.shape[0], Ho, Wo, N)

    p, Ho, Wo = _im2col(x)
    out = _tail(_repack_w(b4_w_t), b4_gamma, b4_beta, p,
                final_wf2.T, final_b, N)
    return out.reshape(N, 1, 1, 1).astype(jnp.float32)


# speedup vs baseline: 14.5709x; 1.0479x over previous
import functools

import jax
import jax.numpy as jnp
from jax.experimental import pallas as pl
from jax.experimental.pallas import tpu as pltpu

_SLOPE = 0.2
_EPS = 1e-5


def _im2col(x):
    C, H, W, N = x.shape
    xp = jnp.pad(x, ((0, 0), (1, 1), (1, 1), (0, 0)))
    Hh, Wh = (H + 2) // 2, (W + 2) // 2
    ph = xp.reshape(C, Hh, 2, Wh, 2, N).transpose(2, 4, 0, 1, 3, 5)
    Ho, Wo = H // 2, W // 2
    taps = [ph[i & 1, j & 1, :, (i >> 1):(i >> 1) + Ho,
               (j >> 1):(j >> 1) + Wo, :]
            for i in range(4) for j in range(4)]
    p = jnp.stack(taps, axis=1)
    return p.reshape(C * 16, Ho * Wo * N), Ho, Wo


def _repack_w(w_t):
    O, K = w_t.shape
    I = K // 16
    return w_t.reshape(O, 4, 4, I).transpose(0, 3, 1, 2).reshape(O, K)


def _pick_tile(ho, won):
    g = 1
    while g < ho and ((g * won) % 128 != 0 or g * won < 2048):
        g *= 2
    return g * won


def _l0_body(w_ref, b_ref, p_ref, o_ref):
    w = w_ref[...].astype(jnp.bfloat16)
    y = jnp.dot(w, p_ref[...], preferred_element_type=jnp.float32)
    y = y + b_ref[...]
    y = jnp.where(y > 0, y, _SLOPE * y)
    o_ref[...] = y.astype(jnp.bfloat16)


def _layer0(w, b, patches, mt):
    C, K = w.shape
    M = patches.shape[1]
    grid = (M // mt,)
    return pl.pallas_call(
        _l0_body,
        out_shape=jax.ShapeDtypeStruct((C, M), jnp.bfloat16),
        grid=grid,
        in_specs=[
            pl.BlockSpec((C, K), lambda m: (0, 0)),
            pl.BlockSpec((C, 1), lambda m: (0, 0)),
            pl.BlockSpec((K, mt), lambda m: (0, m)),
        ],
        out_specs=pl.BlockSpec((C, mt), lambda m: (0, m)),
        compiler_params=pltpu.CompilerParams(
            dimension_semantics=("parallel",)),
    )(w, b, patches)


def _bn_body(w_ref, g_ref, b_ref, p_ref, o_ref, y_scr, sc_ref, sh_ref,
             *, mt, inv_m):
    p_id = pl.program_id(0)
    m = pl.program_id(1)

    @pl.when(p_id == 0)
    def _():
        w = w_ref[...].astype(jnp.bfloat16)
        y_scr[:, pl.ds(m * mt, mt)] = jnp.dot(
            w, p_ref[...], preferred_element_type=jnp.float32)
        o_ref[...] = jnp.zeros_like(o_ref)

    @pl.when((p_id == 1) & (m == 0))
    def _():
        y = y_scr[...]
        mean = jnp.sum(y, axis=1, keepdims=True) * inv_m
        d = y - mean
        var = jnp.sum(d * d, axis=1, keepdims=True) * inv_m
        scale = g_ref[...] * jax.lax.rsqrt(var + _EPS)
        sc_ref[...] = scale
        sh_ref[...] = b_ref[...] - mean * scale

    @pl.when(p_id == 1)
    def _():
        z = y_scr[:, pl.ds(m * mt, mt)] * sc_ref[...] + sh_ref[...]
        z = jnp.where(z > 0, z, _SLOPE * z)
        o_ref[...] = z.astype(jnp.bfloat16)


def _bn_layer(w, gamma, beta, patches, mt):
    C, K = w.shape
    M = patches.shape[1]
    grid = (2, M // mt)
    body = functools.partial(_bn_body, mt=mt, inv_m=1.0 / M)
    return pl.pallas_call(
        body,
        out_shape=jax.ShapeDtypeStruct((C, M), jnp.bfloat16),
        grid=grid,
        in_specs=[
            pl.BlockSpec((C, K), lambda p, m: (0, 0)),
            pl.BlockSpec((C, 1), lambda p, m: (0, 0)),
            pl.BlockSpec((C, 1), lambda p, m: (0, 0)),
            pl.BlockSpec((K, mt),
                         lambda p, m: (0, jnp.where(p == 0, m, 0))),
        ],
        out_specs=pl.BlockSpec((C, mt), lambda p, m: (0, m)),
        scratch_shapes=[
            pltpu.VMEM((C, M), jnp.float32),
            pltpu.VMEM((C, 1), jnp.float32),
            pltpu.VMEM((C, 1), jnp.float32),
        ],
        compiler_params=pltpu.CompilerParams(
            dimension_semantics=("arbitrary", "arbitrary")),
    )(w, gamma, beta, patches)


def _tail_body(w_ref, g_ref, b_ref, p_ref, wf_ref, bf_ref, o_ref, *, n, inv_m):
    w = w_ref[...].astype(jnp.bfloat16)
    y = jnp.dot(w, p_ref[...], preferred_element_type=jnp.float32)
    mean = jnp.sum(y, axis=1, keepdims=True) * inv_m
    d = y - mean
    var = jnp.sum(d * d, axis=1, keepdims=True) * inv_m
    scale = g_ref[...] * jax.lax.rsqrt(var + _EPS)
    z = d * scale + b_ref[...]
    z = jnp.where(z > 0, z, _SLOPE * z)
    acc = bf_ref[...]
    for j in range(16):
        acc = acc + jnp.dot(wf_ref[j:j + 1, :], z[:, j * n:(j + 1) * n],
                            preferred_element_type=jnp.float32)
    o_ref[...] = acc


def _tail(w, gamma, beta, patches, wf_t, bfin, n):
    C, K = w.shape
    M = patches.shape[1]
    body = functools.partial(_tail_body, n=n, inv_m=1.0 / M)
    return pl.pallas_call(
        body,
        out_shape=jax.ShapeDtypeStruct((1, n), jnp.float32),
        grid=(1,),
        in_specs=[
            pl.BlockSpec((C, K), lambda i: (0, 0)),
            pl.BlockSpec((C, 1), lambda i: (0, 0)),
            pl.BlockSpec((C, 1), lambda i: (0, 0)),
            pl.BlockSpec((K, M), lambda i: (0, 0)),
            pl.BlockSpec((16, C), lambda i: (0, 0)),
            pl.BlockSpec((1, 1), lambda i: (0, 0)),
        ],
        out_specs=pl.BlockSpec((1, n), lambda i: (0, 0)),
        compiler_params=pltpu.CompilerParams(
            dimension_semantics=("arbitrary",)),
    )(w, gamma, beta, patches, wf_t, bfin)


def kernel(img, b0_w_t, b0_b, b1_w_t, b1_gamma, b1_beta,
           b2_w_t, b2_gamma, b2_beta, b3_w_t, b3_gamma, b3_beta,
           b4_w_t, b4_gamma, b4_beta, final_wf2, final_b):
    N = img.shape[0]
    x = jnp.transpose(img.astype(jnp.bfloat16), (1, 2, 3, 0))

    p, Ho, Wo = _im2col(x)
    mt = _pick_tile(Ho, Wo * N)
    y = _layer0(_repack_w(b0_w_t), b0_b, p, mt)
    x = y.reshape(8, Ho, Wo, N)

    for w_t, ga, be in ((b1_w_t, b1_gamma, b1_beta),
                        (b2_w_t, b2_gamma, b2_beta),
                        (b3_w_t, b3_gamma, b3_beta)):
        p, Ho, Wo = _im2col(x)
        mt = _pick_tile(Ho, Wo * N)
        y = _bn_layer(_repack_w(w_t), ga, be, p, mt)
        x = y.reshape(w_t.shape[0], Ho, Wo, N)

    p, Ho, Wo = _im2col(x)
    out = _tail(_repack_w(b4_w_t), b4_gamma, b4_beta, p,
                final_wf2.T, final_b, N)
    return out.reshape(N, 1, 1, 1).astype(jnp.float32)
